# baseline (speedup 1.0000x reference)
import jax
import jax.numpy as jnp
from jax.experimental import pallas as pl

_NTOKEN = 1000
_BBLK = 32


def _onehot_body(x_ref, out_ref):
    x = x_ref[...]
    iota = jax.lax.broadcasted_iota(jnp.int32, out_ref.shape, 2)
    out_ref[...] = (x[:, :, None] == iota).astype(jnp.float32)


def _tc_part(x):
    B, L = x.shape
    return pl.pallas_call(
        _onehot_body,
        grid=(B // _BBLK,),
        in_specs=[pl.BlockSpec((_BBLK, L), lambda i: (i, 0))],
        out_specs=pl.BlockSpec((_BBLK, L, _NTOKEN), lambda i: (i, 0, 0)),
        out_shape=jax.ShapeDtypeStruct((B, L, _NTOKEN), jnp.float32),
    )(x)


def kernel(x):
    B = x.shape[0]
    half = B // 2
    return jnp.concatenate([_tc_part(x[:half]), _tc_part(x[half:])], axis=0)
